# Initial kernel scaffold; baseline (speedup 1.0000x reference)
import functools

import jax
import jax.numpy as jnp
from jax import lax
from jax.experimental import pallas as pl
from jax.experimental.pallas import tpu as pltpu
from jax.experimental.pallas import tpu_sc as plsc

N = 10000
F_IN = 128
H = 64
C = 10
G = 64

NC = 2
NS = 16
NW = NC * NS
CHUNK = 128
CHUNKS = 79
E_PAD = NW * CHUNKS * CHUNK
ROWS_PER_TILE = 632
N_PAD = NS * ROWS_PER_TILE
DUMMY = N
DEG_W = 16

_mesh = plsc.VectorSubcoreMesh(core_axis_name="c", subcore_axis_name="s")


@functools.partial(
    pl.kernel,
    out_type=jax.ShapeDtypeStruct((NC, N_PAD, DEG_W), jnp.float32),
    mesh=_mesh,
    scratch_types=[
        pltpu.VMEM((CHUNKS + 1, CHUNK), jnp.int32),
        pltpu.VMEM((CHUNK, DEG_W), jnp.float32),
        pltpu.VMEM_SHARED((N_PAD, DEG_W), jnp.float32),
    ],
)
def _deg_kernel(dstg, zeros, ones, out, dst_v, ones_v, acc_sh):
    c = lax.axis_index("c")
    s = lax.axis_index("s")
    w = c * NS + s
    r0 = s * ROWS_PER_TILE
    pltpu.sync_copy(zeros.at[pl.ds(r0, ROWS_PER_TILE)],
                    acc_sh.at[pl.ds(r0, ROWS_PER_TILE)])
    pltpu.sync_copy(dstg.at[w], dst_v)
    pltpu.sync_copy(ones, ones_v)
    plsc.subcore_barrier()

    def body(j, carry):
        pltpu.sync_copy(ones_v, acc_sh.at[dst_v.at[j]], add=True)
        return carry

    lax.fori_loop(0, CHUNKS, body, 0)
    plsc.subcore_barrier()
    pltpu.sync_copy(acc_sh.at[pl.ds(r0, ROWS_PER_TILE)],
                    out.at[c, pl.ds(r0, ROWS_PER_TILE)])


@functools.partial(
    pl.kernel,
    out_type=jax.ShapeDtypeStruct((NC, N_PAD, H), jnp.float32),
    mesh=_mesh,
    scratch_types=[
        pltpu.VMEM((CHUNKS + 1, CHUNK), jnp.int32),
        pltpu.VMEM((CHUNKS + 1, CHUNK), jnp.int32),
        pltpu.VMEM((CHUNK, H), jnp.float32),
        pltpu.VMEM_SHARED((N_PAD, H), jnp.float32),
        pltpu.SemaphoreType.DMA,
    ],
)
def _edge_kernel(h, srcg, dstg, zeros, out, src_v, dst_v, rows_v, acc_sh, sem):
    c = lax.axis_index("c")
    s = lax.axis_index("s")
    w = c * NS + s
    r0 = s * ROWS_PER_TILE
    pltpu.sync_copy(zeros.at[pl.ds(r0, ROWS_PER_TILE)],
                    acc_sh.at[pl.ds(r0, ROWS_PER_TILE)])
    pltpu.sync_copy(srcg.at[w], src_v)
    pltpu.sync_copy(dstg.at[w], dst_v)
    plsc.subcore_barrier()

    def body(j, carry):
        pltpu.async_copy(h.at[src_v.at[j]], rows_v, sem).wait()
        pltpu.sync_copy(rows_v, acc_sh.at[dst_v.at[j]], add=True)
        return carry

    lax.fori_loop(0, CHUNKS, body, 0)
    plsc.subcore_barrier()
    pltpu.sync_copy(acc_sh.at[pl.ds(r0, ROWS_PER_TILE)],
                    out.at[c, pl.ds(r0, ROWS_PER_TILE)])


def _dinv_of(degp_ref):
    deg = degp_ref[0, :, 0:1] + degp_ref[1, :, 0:1] + 1.0
    return lax.rsqrt(jnp.maximum(deg, 1.0))


_DOT = dict(preferred_element_type=jnp.float32, precision=lax.Precision.HIGHEST)


def _first_body(x_ref, w_ref, degp_ref, out_ref):
    dinv = _dinv_of(degp_ref)
    out_ref[...] = jnp.dot(x_ref[...], w_ref[...], **_DOT) * dinv


def _mid_body(acc_ref, hp_ref, degp_ref, b_ref, w_ref, out_ref):
    dinv = _dinv_of(degp_ref)
    t = dinv * (acc_ref[0] + acc_ref[1] + hp_ref[...]) + b_ref[...]
    t = jnp.maximum(t, 0.0)
    out_ref[...] = jnp.dot(t, w_ref[...], **_DOT) * dinv


def _final_body(acc_ref, hp_ref, degp_ref, b_ref, batch_ref, wc_ref, bc_ref,
                out_ref):
    dinv = _dinv_of(degp_ref)
    h3 = dinv * (acc_ref[0] + acc_ref[1] + hp_ref[...]) + b_ref[...]
    bt = batch_ref[...]
    onehot = (bt[:, None] ==
              lax.broadcasted_iota(jnp.int32, (N_PAD, G), 1)).astype(jnp.float32)
    sums = lax.dot_general(onehot, h3, (((0,), (0,)), ((), ())), **_DOT)
    cnt = jnp.sum(onehot, axis=0)
    pooled = sums / jnp.maximum(cnt, 1.0)[:, None]
    out_ref[...] = jnp.dot(pooled, wc_ref[...], **_DOT) + bc_ref[...]


_first_mm = pl.pallas_call(
    _first_body, out_shape=jax.ShapeDtypeStruct((N_PAD, H), jnp.float32))
_mid_mm = pl.pallas_call(
    _mid_body, out_shape=jax.ShapeDtypeStruct((N_PAD, H), jnp.float32))
_final_mm = pl.pallas_call(
    _final_body, out_shape=jax.ShapeDtypeStruct((G, C), jnp.float32))


def kernel(x, edge_index, batch, W1, b1, W2, b2, W3, b3, Wc, bc):
    src = edge_index[0].astype(jnp.int32)
    dst = edge_index[1].astype(jnp.int32)
    e = src.shape[0]
    srcp = jnp.concatenate(
        [src, jnp.zeros((E_PAD - e,), jnp.int32)]).reshape(NW, CHUNKS, CHUNK)
    dstp = jnp.concatenate(
        [dst, jnp.full((E_PAD - e,), DUMMY, jnp.int32)]).reshape(NW, CHUNKS, CHUNK)
    srcg = jnp.concatenate(
        [srcp, jnp.zeros((NW, 1, CHUNK), jnp.int32)], axis=1)
    dstg = jnp.concatenate(
        [dstp, jnp.full((NW, 1, CHUNK), DUMMY, jnp.int32)], axis=1)

    xp = jnp.zeros((N_PAD, F_IN), jnp.float32).at[:N].set(x)
    batchp = jnp.full((N_PAD,), G, jnp.int32).at[:N].set(batch.astype(jnp.int32))
    zeros_deg = jnp.zeros((N_PAD, DEG_W), jnp.float32)
    ones_deg = jnp.ones((CHUNK, DEG_W), jnp.float32)
    zeros_acc = jnp.zeros((N_PAD, H), jnp.float32)

    degp = _deg_kernel(dstg, zeros_deg, ones_deg)
    h1 = _first_mm(xp, W1, degp)
    acc1 = _edge_kernel(h1, srcg, dstg, zeros_acc)
    h2 = _mid_mm(acc1, h1, degp, b1, W2)
    acc2 = _edge_kernel(h2, srcg, dstg, zeros_acc)
    h3 = _mid_mm(acc2, h2, degp, b2, W3)
    acc3 = _edge_kernel(h3, srcg, dstg, zeros_acc)
    return _final_mm(acc3, h3, degp, b3, batchp, Wc, bc)


# speedup vs baseline: 17.3303x; 17.3303x over previous
import functools

import jax
import jax.numpy as jnp
from jax import lax
from jax.experimental import pallas as pl
from jax.experimental.pallas import tpu as pltpu
from jax.experimental.pallas import tpu_sc as plsc

N = 10000
F_IN = 128
H = 64
C = 10
G = 64

NC = 2
NS = 16
NW = NC * NS
CHUNK = 128
CHUNKS = 79
E_PAD = NW * CHUNKS * CHUNK
ROWS_PER_TILE = 632
N_PAD = NS * ROWS_PER_TILE
DUMMY = N
DEG_W = 16

_mesh = plsc.VectorSubcoreMesh(core_axis_name="c", subcore_axis_name="s",
                               num_cores=NC, num_subcores=NS)


_SC_PARAMS = pltpu.CompilerParams(use_tc_tiling_on_sc=False)


@functools.partial(
    pl.kernel,
    out_type=jax.ShapeDtypeStruct((NC, N_PAD, DEG_W), jnp.float32),
    mesh=_mesh,
    compiler_params=_SC_PARAMS,
    scratch_types=[
        pltpu.VMEM((CHUNKS + 1, CHUNK), jnp.int32),
        pltpu.VMEM((CHUNK, DEG_W), jnp.float32),
        pltpu.VMEM_SHARED((N_PAD, DEG_W), jnp.float32),
    ],
)
def _deg_kernel(dstg, zeros, ones, out, dst_v, ones_v, acc_sh):
    c = lax.axis_index("c")
    s = lax.axis_index("s")
    w = c * NS + s
    r0 = s * ROWS_PER_TILE
    pltpu.sync_copy(zeros.at[pl.ds(r0, ROWS_PER_TILE)],
                    acc_sh.at[pl.ds(r0, ROWS_PER_TILE)])
    pltpu.sync_copy(dstg.at[w], dst_v)
    pltpu.sync_copy(ones, ones_v)
    plsc.subcore_barrier()

    def body(j, carry):
        pltpu.sync_copy(ones_v, acc_sh.at[dst_v.at[j]], add=True)
        return carry

    lax.fori_loop(0, CHUNKS, body, 0)
    plsc.subcore_barrier()
    pltpu.sync_copy(acc_sh.at[pl.ds(r0, ROWS_PER_TILE)],
                    out.at[c, pl.ds(r0, ROWS_PER_TILE)])


@functools.partial(
    pl.kernel,
    out_type=jax.ShapeDtypeStruct((NC, N_PAD, H), jnp.float32),
    mesh=_mesh,
    compiler_params=_SC_PARAMS,
    scratch_types=[
        pltpu.VMEM((CHUNKS + 1, CHUNK), jnp.int32),
        pltpu.VMEM((CHUNKS + 1, CHUNK), jnp.int32),
        pltpu.VMEM((CHUNK, H), jnp.float32),
        pltpu.VMEM_SHARED((N_PAD, H), jnp.float32),
        pltpu.SemaphoreType.DMA,
    ],
)
def _edge_kernel(h, srcg, dstg, zeros, out, src_v, dst_v, rows_v, acc_sh, sem):
    c = lax.axis_index("c")
    s = lax.axis_index("s")
    w = c * NS + s
    r0 = s * ROWS_PER_TILE
    pltpu.sync_copy(zeros.at[pl.ds(r0, ROWS_PER_TILE)],
                    acc_sh.at[pl.ds(r0, ROWS_PER_TILE)])
    pltpu.sync_copy(srcg.at[w], src_v)
    pltpu.sync_copy(dstg.at[w], dst_v)
    plsc.subcore_barrier()

    def body(j, carry):
        pltpu.async_copy(h.at[src_v.at[j]], rows_v, sem).wait()
        pltpu.sync_copy(rows_v, acc_sh.at[dst_v.at[j]], add=True)
        return carry

    lax.fori_loop(0, CHUNKS, body, 0)
    plsc.subcore_barrier()
    pltpu.sync_copy(acc_sh.at[pl.ds(r0, ROWS_PER_TILE)],
                    out.at[c, pl.ds(r0, ROWS_PER_TILE)])


def _dinv_of(degp_ref):
    deg = degp_ref[0, :, 0:1] + degp_ref[1, :, 0:1] + 1.0
    return lax.rsqrt(jnp.maximum(deg, 1.0))


_DOT = dict(preferred_element_type=jnp.float32, precision=lax.Precision.HIGHEST)


def _first_body(x_ref, w_ref, degp_ref, out_ref):
    dinv = _dinv_of(degp_ref)
    out_ref[...] = jnp.dot(x_ref[...], w_ref[...], **_DOT) * dinv


def _mid_body(acc_ref, hp_ref, degp_ref, b_ref, w_ref, out_ref):
    dinv = _dinv_of(degp_ref)
    t = dinv * (acc_ref[0] + acc_ref[1] + hp_ref[...]) + b_ref[...]
    t = jnp.maximum(t, 0.0)
    out_ref[...] = jnp.dot(t, w_ref[...], **_DOT) * dinv


def _final_body(acc_ref, hp_ref, degp_ref, b_ref, batch_ref, wc_ref, bc_ref,
                out_ref):
    dinv = _dinv_of(degp_ref)
    h3 = dinv * (acc_ref[0] + acc_ref[1] + hp_ref[...]) + b_ref[...]
    bt = batch_ref[...]
    onehot = (bt[:, None] ==
              lax.broadcasted_iota(jnp.int32, (N_PAD, G), 1)).astype(jnp.float32)
    sums = lax.dot_general(onehot, h3, (((0,), (0,)), ((), ())), **_DOT)
    cnt = jnp.sum(onehot, axis=0)
    pooled = sums / jnp.maximum(cnt, 1.0)[:, None]
    out_ref[...] = jnp.dot(pooled, wc_ref[...], **_DOT) + bc_ref[...]


_first_mm = pl.pallas_call(
    _first_body, out_shape=jax.ShapeDtypeStruct((N_PAD, H), jnp.float32))
_mid_mm = pl.pallas_call(
    _mid_body, out_shape=jax.ShapeDtypeStruct((N_PAD, H), jnp.float32))
_final_mm = pl.pallas_call(
    _final_body, out_shape=jax.ShapeDtypeStruct((G, C), jnp.float32))


def kernel(x, edge_index, batch, W1, b1, W2, b2, W3, b3, Wc, bc):
    src = edge_index[0].astype(jnp.int32)
    dst = edge_index[1].astype(jnp.int32)
    e = src.shape[0]
    srcp = jnp.concatenate(
        [src, jnp.zeros((E_PAD - e,), jnp.int32)]).reshape(NW, CHUNKS, CHUNK)
    dstp = jnp.concatenate(
        [dst, jnp.full((E_PAD - e,), DUMMY, jnp.int32)]).reshape(NW, CHUNKS, CHUNK)
    srcg = jnp.concatenate(
        [srcp, jnp.zeros((NW, 1, CHUNK), jnp.int32)], axis=1)
    dstg = jnp.concatenate(
        [dstp, jnp.full((NW, 1, CHUNK), DUMMY, jnp.int32)], axis=1)

    xp = jnp.zeros((N_PAD, F_IN), jnp.float32).at[:N].set(x)
    batchp = jnp.full((N_PAD,), G, jnp.int32).at[:N].set(batch.astype(jnp.int32))
    zeros_deg = jnp.zeros((N_PAD, DEG_W), jnp.float32)
    ones_deg = jnp.ones((CHUNK, DEG_W), jnp.float32)
    zeros_acc = jnp.zeros((N_PAD, H), jnp.float32)

    degp = _deg_kernel(dstg, zeros_deg, ones_deg)
    h1 = _first_mm(xp, W1, degp)
    acc1 = _edge_kernel(h1, srcg, dstg, zeros_acc)
    h2 = _mid_mm(acc1, h1, degp, b1, W2)
    acc2 = _edge_kernel(h2, srcg, dstg, zeros_acc)
    h3 = _mid_mm(acc2, h2, degp, b2, W3)
    acc3 = _edge_kernel(h3, srcg, dstg, zeros_acc)
    return _final_mm(acc3, h3, degp, b3, batchp, Wc, bc)
